# baseline (speedup 1.0000x reference)
import jax
import jax.numpy as jnp
from jax import lax
from jax.experimental import pallas as pl
from jax.experimental.pallas import tpu as pltpu
from jax.experimental.pallas import tpu_sc as plsc

_U = 6
_L = 16


def _sc_sel_kernel(mask_hbm, sel_hbm, nsel_hbm, mask_v, sel_v, nsel_v):
    ci = lax.axis_index("c")
    si = lax.axis_index("s")

    @pl.when((ci == 0) & (si == 0))
    def _only_tile0():
        c = mask_v.shape[0]
        pltpu.sync_copy(mask_hbm, mask_v)
        for g in range(c // _L):
            sel_v[pl.ds(g * _L, _L)] = jnp.zeros((_L,), jnp.int32)
        base = jnp.int32(0)
        for g in range(c // _L):
            v = mask_v[pl.ds(g * _L, _L)]
            nz = v != 0.0
            nzi = jnp.where(nz, jnp.int32(1), jnp.int32(0))
            cs = plsc.cumsum(nzi)
            pos = cs - nzi + base
            vals = lax.iota(jnp.int32, _L) + jnp.int32(g * _L)
            plsc.store_scatter(sel_v, [pos], vals, mask=nz)
            base = base + jnp.sum(nzi)
        nsel_v[...] = jnp.broadcast_to(base, (_L,))
        pltpu.sync_copy(sel_v, sel_hbm)
        pltpu.sync_copy(nsel_v, nsel_hbm)


def _compute_sel_sc(indexes):
    c = indexes.shape[0]
    mesh = plsc.VectorSubcoreMesh(core_axis_name="c", subcore_axis_name="s")
    import functools

    @functools.partial(
        pl.kernel,
        mesh=mesh,
        out_type=(
            jax.ShapeDtypeStruct((c,), jnp.int32),
            jax.ShapeDtypeStruct((_L,), jnp.int32),
        ),
        scratch_types=[
            pltpu.VMEM((c,), jnp.float32),
            pltpu.VMEM((c,), jnp.int32),
            pltpu.VMEM((_L,), jnp.int32),
        ],
        compiler_params=pltpu.CompilerParams(needs_layout_passes=False),
    )
    def run(mask_hbm, sel_hbm, nsel_hbm, mask_v, sel_v, nsel_v):
        _sc_sel_kernel(mask_hbm, sel_hbm, nsel_hbm, mask_v, sel_v, nsel_v)

    sel, nsel = run(indexes)
    return sel, nsel[0:1]


def _copy_kernel(sel_ref, nsel_ref, *refs):
    del sel_ref
    ins = refs[:_U]
    out_ref = refs[_U]
    k = pl.program_id(0)
    nsel = nsel_ref[0]
    for u in range(_U):
        j = _U * k + u

        @pl.when(j < nsel)
        def _valid(u=u):
            out_ref[:, u : u + 1] = ins[u][...]

        @pl.when(j >= nsel)
        def _invalid(u=u):
            out_ref[:, u : u + 1] = jnp.full_like(ins[u], jnp.nan)


def kernel(input_tensor, indexes):
    n, c, h, w = input_tensor.shape

    sel, nsel = _compute_sel_sc(indexes)

    def _in_spec(u):
        return pl.BlockSpec(
            (n, 1, h, w),
            lambda k, sel_ref, nsel_ref: (0, sel_ref[_U * k + u], 0, 0),
        )

    grid_spec = pltpu.PrefetchScalarGridSpec(
        num_scalar_prefetch=2,
        grid=(c // _U,),
        in_specs=[_in_spec(u) for u in range(_U)],
        out_specs=pl.BlockSpec(
            (n, _U, h, w), lambda k, sel_ref, nsel_ref: (0, k, 0, 0)
        ),
    )
    return pl.pallas_call(
        _copy_kernel,
        grid_spec=grid_spec,
        out_shape=jax.ShapeDtypeStruct((n, c, h, w), input_tensor.dtype),
        compiler_params=pltpu.CompilerParams(
            dimension_semantics=("parallel",),
        ),
    )(sel, nsel, *([input_tensor] * _U))


# speedup vs baseline: 1.1553x; 1.1553x over previous
import jax
import jax.numpy as jnp
from jax.experimental import pallas as pl
from jax.experimental.pallas import tpu as pltpu

_U = 8


def _sel_kernel(mask_ref, sel_ref, nsel_ref):
    c = mask_ref.shape[-1]
    nz = mask_ref[...] != 0.0
    nzi = nz.astype(jnp.int32)
    row = jax.lax.broadcasted_iota(jnp.int32, (c, c), 0)
    col = jax.lax.broadcasted_iota(jnp.int32, (c, c), 1)
    rank = jnp.sum((nz & (col < row)).astype(jnp.int32), axis=1)
    m = nz & (jnp.broadcast_to(rank[None, :], (c, c)) == row)
    sel = jnp.sum(jnp.where(m, col, 0), axis=1)
    sel_ref[...] = sel.reshape(1, c)
    nsel_ref[...] = jnp.sum(nzi, axis=-1, keepdims=True)


def _copy_kernel(sel_ref, nsel_ref, *refs):
    del sel_ref
    ins = refs[:_U]
    out_ref = refs[_U]
    k = pl.program_id(0)
    nsel = nsel_ref[0]
    for u in range(_U):
        j = _U * k + u

        @pl.when(j < nsel)
        def _valid(u=u):
            out_ref[:, u : u + 1] = ins[u][...]

        @pl.when(j >= nsel)
        def _invalid(u=u):
            out_ref[:, u : u + 1] = jnp.full_like(ins[u], jnp.nan)


def kernel(input_tensor, indexes):
    n, c, h, w = input_tensor.shape

    sel, nsel = pl.pallas_call(
        _sel_kernel,
        out_shape=(
            jax.ShapeDtypeStruct((1, c), jnp.int32),
            jax.ShapeDtypeStruct((1, 1), jnp.int32),
        ),
    )(indexes.reshape(1, c))
    sel = sel.reshape(c)
    nsel = nsel.reshape(1)

    def _in_spec(u):
        return pl.BlockSpec(
            (n, 1, h, w),
            lambda k, sel_ref, nsel_ref: (0, sel_ref[_U * k + u], 0, 0),
        )

    grid_spec = pltpu.PrefetchScalarGridSpec(
        num_scalar_prefetch=2,
        grid=(c // _U,),
        in_specs=[_in_spec(u) for u in range(_U)],
        out_specs=pl.BlockSpec(
            (n, _U, h, w), lambda k, sel_ref, nsel_ref: (0, k, 0, 0)
        ),
    )
    return pl.pallas_call(
        _copy_kernel,
        grid_spec=grid_spec,
        out_shape=jax.ShapeDtypeStruct((n, c, h, w), input_tensor.dtype),
        compiler_params=pltpu.CompilerParams(
            dimension_semantics=("parallel",),
            vmem_limit_bytes=64 * 1024 * 1024,
        ),
    )(sel, nsel, *([input_tensor] * _U))
